# initial kernel scaffold (unmeasured)
import jax
import jax.numpy as jnp
from jax import lax
from jax.experimental import pallas as pl
from jax.experimental.pallas import tpu as pltpu

N_DEV = 32
SQ = 1024
D = 1024
HQ_LOCAL = 8
DH = 128
BLK = 64
SCALE = 0.08838834764831843
CHUNK = SQ // N_DEV


def _compute_body(x_ref, wq_ref, k_ref, v_ref, wo_ref, out_ref):
    Q = jnp.dot(x_ref[:, :], wq_ref[:, :], preferred_element_type=jnp.float32)
    qb = lax.broadcasted_iota(jnp.int32, (SQ, SQ), 0) // BLK
    kb = lax.broadcasted_iota(jnp.int32, (SQ, SQ), 1) // BLK
    mask = kb <= qb
    K = k_ref[:, :]
    V = v_ref[:, :]
    ctx_parts = []
    for h in range(HQ_LOCAL):
        q_h = Q[:, h * DH:(h + 1) * DH]
        k_h = K[:, h * DH:(h + 1) * DH]
        v_h = V[:, h * DH:(h + 1) * DH]
        s = lax.dot_general(
            q_h, k_h, (((1,), (1,)), ((), ())),
            preferred_element_type=jnp.float32,
        ) * SCALE
        s = jnp.where(mask, s, -1e9)
        m = jnp.max(s, axis=1, keepdims=True)
        w = jnp.exp(s - m)
        w = w / jnp.sum(w, axis=1, keepdims=True)
        ctx_parts.append(
            jnp.dot(w, v_h, preferred_element_type=jnp.float32)
        )
    ctx = jnp.concatenate(ctx_parts, axis=1)
    out_ref[:, :] = jnp.dot(ctx, wo_ref[:, :], preferred_element_type=jnp.float32)


def _allreduce_body(partial_ref, out_ref, rs_buf,
                    rs_send, rs_recv, ag_send, ag_recv):
    my = lax.axis_index("i")

    rs_descs = []
    for k in range(1, N_DEV):
        peer = lax.rem(my + k, N_DEV)
        rdma = pltpu.make_async_remote_copy(
            src_ref=partial_ref.at[pl.ds(peer * CHUNK, CHUNK), :],
            dst_ref=rs_buf.at[k - 1],
            send_sem=rs_send.at[k - 1],
            recv_sem=rs_recv.at[k - 1],
            device_id=(peer,),
            device_id_type=pl.DeviceIdType.MESH,
        )
        rdma.start()
        rs_descs.append(rdma)
    for r in rs_descs:
        r.wait_recv()

    red = partial_ref[pl.ds(my * CHUNK, CHUNK), :] + jnp.sum(rs_buf[:, :, :], axis=0)
    out_ref[pl.ds(my * CHUNK, CHUNK), :] = red

    ag_descs = []
    for k in range(1, N_DEV):
        peer = lax.rem(my + k, N_DEV)
        rdma = pltpu.make_async_remote_copy(
            src_ref=out_ref.at[pl.ds(my * CHUNK, CHUNK), :],
            dst_ref=out_ref.at[pl.ds(my * CHUNK, CHUNK), :],
            send_sem=ag_send.at[k - 1],
            recv_sem=ag_recv.at[k - 1],
            device_id=(peer,),
            device_id_type=pl.DeviceIdType.MESH,
        )
        rdma.start()
        ag_descs.append(rdma)
    for r in ag_descs:
        r.wait_recv()
    for r in rs_descs:
        r.wait_send()
    for r in ag_descs:
        r.wait_send()


def kernel(x, Wq, K_ext, V_ext, Wo):
    my = lax.axis_index("i")

    k2 = lax.dynamic_slice(
        K_ext[0].reshape(SQ, 256 * DH), (0, my * D), (SQ, D))
    v2 = lax.dynamic_slice(
        V_ext[0].reshape(SQ, 256 * DH), (0, my * D), (SQ, D))

    partial = pl.pallas_call(
        _compute_body,
        out_shape=jax.ShapeDtypeStruct((SQ, D), jnp.float32),
        in_specs=[pl.BlockSpec(memory_space=pltpu.VMEM)] * 5,
        out_specs=pl.BlockSpec(memory_space=pltpu.VMEM),
    )(x[0], Wq, k2, v2, Wo)

    out = pl.pallas_call(
        _allreduce_body,
        out_shape=jax.ShapeDtypeStruct((SQ, D), jnp.float32),
        in_specs=[pl.BlockSpec(memory_space=pltpu.VMEM)],
        out_specs=pl.BlockSpec(memory_space=pltpu.VMEM),
        scratch_shapes=[
            pltpu.VMEM((N_DEV - 1, CHUNK, D), jnp.float32),
            pltpu.SemaphoreType.DMA((N_DEV - 1,)),
            pltpu.SemaphoreType.DMA((N_DEV - 1,)),
            pltpu.SemaphoreType.DMA((N_DEV - 1,)),
            pltpu.SemaphoreType.DMA((N_DEV - 1,)),
        ],
        compiler_params=pltpu.CompilerParams(collective_id=0),
    )(partial)

    return out.reshape(1, SQ, D)


# baseline (device time: 3531685 ns/iter reference)
import jax
import jax.numpy as jnp
from jax import lax
from jax.experimental import pallas as pl
from jax.experimental.pallas import tpu as pltpu

N_DEV = 32
SQ = 1024
D = 1024
HQ_LOCAL = 8
DH = 128
BLK = 64
SCALE = 0.08838834764831843
CHUNK = SQ // N_DEV


def _compute_body(x_ref, wq_ref, k_ref, v_ref, wo_ref, out_ref):
    Q = jnp.dot(x_ref[:, :], wq_ref[:, :], preferred_element_type=jnp.float32)
    qb = lax.broadcasted_iota(jnp.int32, (SQ, SQ), 0) // BLK
    kb = lax.broadcasted_iota(jnp.int32, (SQ, SQ), 1) // BLK
    mask = kb <= qb
    K = k_ref[:, :]
    V = v_ref[:, :]
    ctx_parts = []
    for h in range(HQ_LOCAL):
        q_h = Q[:, h * DH:(h + 1) * DH]
        k_h = K[:, h * DH:(h + 1) * DH]
        v_h = V[:, h * DH:(h + 1) * DH]
        s = lax.dot_general(
            q_h, k_h, (((1,), (1,)), ((), ())),
            preferred_element_type=jnp.float32,
        ) * SCALE
        s = jnp.where(mask, s, -1e9)
        m = jnp.max(s, axis=1, keepdims=True)
        w = jnp.exp(s - m)
        w = w / jnp.sum(w, axis=1, keepdims=True)
        ctx_parts.append(
            jnp.dot(w, v_h, preferred_element_type=jnp.float32)
        )
    ctx = jnp.concatenate(ctx_parts, axis=1)
    out_ref[:, :] = jnp.dot(ctx, wo_ref[:, :], preferred_element_type=jnp.float32)


def _allreduce_body(partial_ref, out_ref, rs_buf,
                    rs_send, rs_recv, ag_send, ag_recv):
    my = lax.axis_index("i")

    rs_descs = []
    for k in range(1, N_DEV):
        peer = lax.rem(my + k, N_DEV)
        rdma = pltpu.make_async_remote_copy(
            src_ref=partial_ref.at[pl.ds(peer * CHUNK, CHUNK), :],
            dst_ref=rs_buf.at[k - 1],
            send_sem=rs_send.at[k - 1],
            recv_sem=rs_recv.at[k - 1],
            device_id=(peer,),
            device_id_type=pl.DeviceIdType.MESH,
        )
        rdma.start()
        rs_descs.append(rdma)
    for r in rs_descs:
        r.wait_recv()

    red = partial_ref[pl.ds(my * CHUNK, CHUNK), :] + jnp.sum(rs_buf[:, :, :], axis=0)
    out_ref[pl.ds(my * CHUNK, CHUNK), :] = red

    ag_descs = []
    for k in range(1, N_DEV):
        peer = lax.rem(my + k, N_DEV)
        rdma = pltpu.make_async_remote_copy(
            src_ref=out_ref.at[pl.ds(my * CHUNK, CHUNK), :],
            dst_ref=out_ref.at[pl.ds(my * CHUNK, CHUNK), :],
            send_sem=ag_send.at[k - 1],
            recv_sem=ag_recv.at[k - 1],
            device_id=(peer,),
            device_id_type=pl.DeviceIdType.MESH,
        )
        rdma.start()
        ag_descs.append(rdma)
    for r in ag_descs:
        r.wait_recv()
    for r in rs_descs:
        r.wait_send()
    for r in ag_descs:
        r.wait_send()


def kernel(x, Wq, K_ext, V_ext, Wo):
    my = lax.axis_index("i")

    k2 = lax.dynamic_slice(
        K_ext[0].reshape(SQ, 256 * DH), (0, my * D), (SQ, D))
    v2 = lax.dynamic_slice(
        V_ext[0].reshape(SQ, 256 * DH), (0, my * D), (SQ, D))

    partial = pl.pallas_call(
        _compute_body,
        out_shape=jax.ShapeDtypeStruct((SQ, D), jnp.float32),
        in_specs=[pl.BlockSpec(memory_space=pltpu.VMEM)] * 5,
        out_specs=pl.BlockSpec(memory_space=pltpu.VMEM),
    )(x[0], Wq, k2, v2, Wo)

    out = pl.pallas_call(
        _allreduce_body,
        out_shape=jax.ShapeDtypeStruct((SQ, D), jnp.float32),
        in_specs=[pl.BlockSpec(memory_space=pltpu.VMEM)],
        out_specs=pl.BlockSpec(memory_space=pltpu.VMEM),
        scratch_shapes=[
            pltpu.VMEM((N_DEV - 1, CHUNK, D), jnp.float32),
            pltpu.SemaphoreType.DMA((N_DEV - 1,)),
            pltpu.SemaphoreType.DMA((N_DEV - 1,)),
            pltpu.SemaphoreType.DMA((N_DEV - 1,)),
            pltpu.SemaphoreType.DMA((N_DEV - 1,)),
        ],
    )(partial)

    return out.reshape(1, SQ, D)


# device time: 149525 ns/iter; 23.6194x vs baseline; 23.6194x over previous
import jax
import jax.numpy as jnp
from jax import lax
from jax.experimental import pallas as pl
from jax.experimental.pallas import tpu as pltpu

N_DEV = 32
SQ = 1024
D = 1024
HQ = 256
HQ_LOCAL = 8
DH = 128
BLK = 64
SCALE = 0.08838834764831843
CHUNK = SQ // N_DEV


def _compute_body(x_ref, wq_ref, k_any, v_any, wo_ref, out_ref,
                  k_vmem, v_vmem, copy_sems):
    my = lax.axis_index("i")
    h0 = my * HQ_LOCAL

    copies = []
    for h in range(HQ_LOCAL):
        ck = pltpu.make_async_copy(
            k_any.at[:, h0 + h, :], k_vmem.at[h], copy_sems.at[h])
        cv = pltpu.make_async_copy(
            v_any.at[:, h0 + h, :], v_vmem.at[h], copy_sems.at[HQ_LOCAL + h])
        ck.start()
        cv.start()
        copies.append(ck)
        copies.append(cv)

    Q = jnp.dot(x_ref[:, :], wq_ref[:, :], preferred_element_type=jnp.float32)
    qb = lax.broadcasted_iota(jnp.int32, (SQ, SQ), 0) // BLK
    kb = lax.broadcasted_iota(jnp.int32, (SQ, SQ), 1) // BLK
    mask = kb <= qb

    for c in copies:
        c.wait()

    ctx_parts = []
    for h in range(HQ_LOCAL):
        q_h = Q[:, h * DH:(h + 1) * DH]
        s = lax.dot_general(
            q_h, k_vmem[h], (((1,), (1,)), ((), ())),
            preferred_element_type=jnp.float32,
        ) * SCALE
        s = jnp.where(mask, s, -1e9)
        m = jnp.max(s, axis=1, keepdims=True)
        w = jnp.exp(s - m)
        w = w / jnp.sum(w, axis=1, keepdims=True)
        ctx_parts.append(
            jnp.dot(w, v_vmem[h], preferred_element_type=jnp.float32)
        )
    ctx = jnp.concatenate(ctx_parts, axis=1)
    out_ref[:, :] = jnp.dot(ctx, wo_ref[:, :], preferred_element_type=jnp.float32)


def _allreduce_body(partial_ref, out_ref, rs_buf,
                    rs_send, rs_recv, ag_send, ag_recv):
    my = lax.axis_index("i")

    rs_descs = []
    for k in range(1, N_DEV):
        peer = lax.rem(my + k, N_DEV)
        rdma = pltpu.make_async_remote_copy(
            src_ref=partial_ref.at[pl.ds(peer * CHUNK, CHUNK), :],
            dst_ref=rs_buf.at[k - 1],
            send_sem=rs_send.at[k - 1],
            recv_sem=rs_recv.at[k - 1],
            device_id=(peer,),
            device_id_type=pl.DeviceIdType.MESH,
        )
        rdma.start()
        rs_descs.append(rdma)
    for r in rs_descs:
        r.wait_recv()

    red = partial_ref[pl.ds(my * CHUNK, CHUNK), :] + jnp.sum(rs_buf[:, :, :], axis=0)
    out_ref[pl.ds(my * CHUNK, CHUNK), :] = red

    ag_descs = []
    for k in range(1, N_DEV):
        peer = lax.rem(my + k, N_DEV)
        rdma = pltpu.make_async_remote_copy(
            src_ref=out_ref.at[pl.ds(my * CHUNK, CHUNK), :],
            dst_ref=out_ref.at[pl.ds(my * CHUNK, CHUNK), :],
            send_sem=ag_send.at[k - 1],
            recv_sem=ag_recv.at[k - 1],
            device_id=(peer,),
            device_id_type=pl.DeviceIdType.MESH,
        )
        rdma.start()
        ag_descs.append(rdma)
    for r in ag_descs:
        r.wait_recv()
    for r in rs_descs:
        r.wait_send()
    for r in ag_descs:
        r.wait_send()


def kernel(x, Wq, K_ext, V_ext, Wo):
    partial = pl.pallas_call(
        _compute_body,
        out_shape=jax.ShapeDtypeStruct((SQ, D), jnp.float32),
        in_specs=[
            pl.BlockSpec(memory_space=pltpu.VMEM),
            pl.BlockSpec(memory_space=pltpu.VMEM),
            pl.BlockSpec(memory_space=pl.ANY),
            pl.BlockSpec(memory_space=pl.ANY),
            pl.BlockSpec(memory_space=pltpu.VMEM),
        ],
        out_specs=pl.BlockSpec(memory_space=pltpu.VMEM),
        scratch_shapes=[
            pltpu.VMEM((HQ_LOCAL, SQ, DH), jnp.float32),
            pltpu.VMEM((HQ_LOCAL, SQ, DH), jnp.float32),
            pltpu.SemaphoreType.DMA((2 * HQ_LOCAL,)),
        ],
    )(x[0], Wq, K_ext[0], V_ext[0], Wo)

    out = pl.pallas_call(
        _allreduce_body,
        out_shape=jax.ShapeDtypeStruct((SQ, D), jnp.float32),
        in_specs=[pl.BlockSpec(memory_space=pltpu.VMEM)],
        out_specs=pl.BlockSpec(memory_space=pltpu.VMEM),
        scratch_shapes=[
            pltpu.VMEM((N_DEV - 1, CHUNK, D), jnp.float32),
            pltpu.SemaphoreType.DMA((N_DEV - 1,)),
            pltpu.SemaphoreType.DMA((N_DEV - 1,)),
            pltpu.SemaphoreType.DMA((N_DEV - 1,)),
            pltpu.SemaphoreType.DMA((N_DEV - 1,)),
        ],
    )(partial)

    return out.reshape(1, SQ, D)


# device time: 142464 ns/iter; 24.7900x vs baseline; 1.0496x over previous
import jax
import jax.numpy as jnp
from jax import lax
from jax.experimental import pallas as pl
from jax.experimental.pallas import tpu as pltpu

N_DEV = 32
SQ = 1024
D = 1024
HQ = 256
HQ_LOCAL = 8
DH = 128
BLK = 64
SCALE = 0.08838834764831843
CHUNK = SQ // N_DEV
NBLK = 4
RB = SQ // NBLK
CPB = RB // CHUNK


def _rs_desc(pbuf, rs_buf, rs_send, rs_recv, my, c):
    return pltpu.make_async_remote_copy(
        src_ref=pbuf.at[pl.ds(c * CHUNK, CHUNK), :],
        dst_ref=rs_buf.at[my],
        send_sem=rs_send.at[c],
        recv_sem=rs_recv.at[my],
        device_id=(c,),
        device_id_type=pl.DeviceIdType.MESH,
    )


def _fused_body(x_ref, wq_ref, k_any, v_any, wo_ref, out_ref,
                k_vmem, v_vmem, pbuf, rs_buf,
                kv_sems, rs_send, rs_recv, ag_send, ag_recv):
    my = lax.axis_index("i")
    h0 = my * HQ_LOCAL

    kv_copies = []
    for h in range(HQ_LOCAL):
        ck = pltpu.make_async_copy(
            k_any.at[:, h0 + h, :], k_vmem.at[h], kv_sems.at[h])
        cv = pltpu.make_async_copy(
            v_any.at[:, h0 + h, :], v_vmem.at[h], kv_sems.at[HQ_LOCAL + h])
        ck.start()
        cv.start()
        kv_copies.append(ck)
        kv_copies.append(cv)

    Q = jnp.dot(x_ref[:, :], wq_ref[:, :], preferred_element_type=jnp.float32)
    Wo = wo_ref[:, :]

    for c in kv_copies:
        c.wait()

    for b in range(NBLK):
        r0 = b * RB
        ncol = (b + 1) * RB
        qblk = (r0 + lax.broadcasted_iota(jnp.int32, (RB, ncol), 0)) // BLK
        kblk = lax.broadcasted_iota(jnp.int32, (RB, ncol), 1) // BLK
        mask = kblk <= qblk
        ctx_parts = []
        for h in range(HQ_LOCAL):
            q_h = Q[r0:r0 + RB, h * DH:(h + 1) * DH]
            s = lax.dot_general(
                q_h, k_vmem[h][:ncol, :], (((1,), (1,)), ((), ())),
                preferred_element_type=jnp.float32,
            ) * SCALE
            s = jnp.where(mask, s, -1e9)
            m = jnp.max(s, axis=1, keepdims=True)
            w = jnp.exp(s - m)
            w = w / jnp.sum(w, axis=1, keepdims=True)
            ctx_parts.append(
                jnp.dot(w, v_vmem[h][:ncol, :],
                        preferred_element_type=jnp.float32))
        ctx_b = jnp.concatenate(ctx_parts, axis=1)
        pbuf[pl.ds(r0, RB), :] = jnp.dot(
            ctx_b, Wo, preferred_element_type=jnp.float32)

        for j in range(CPB):
            c = b * CPB + j

            @pl.when(my != c)
            def _(c=c):
                _rs_desc(pbuf, rs_buf, rs_send, rs_recv, my, c).start()

    pbuf_my = pbuf[pl.ds(my * CHUNK, CHUNK), :]
    rs_buf[pl.ds(my, 1)] = pbuf_my[None, :, :]

    for k in range(1, N_DEV):
        s = lax.rem(my + k, N_DEV)
        pltpu.make_async_remote_copy(
            src_ref=pbuf.at[pl.ds(0, CHUNK), :],
            dst_ref=rs_buf.at[s],
            send_sem=rs_send.at[0],
            recv_sem=rs_recv.at[s],
            device_id=(s,),
            device_id_type=pl.DeviceIdType.MESH,
        ).wait_recv()

    red = jnp.sum(rs_buf[:, :, :], axis=0)
    out_ref[pl.ds(my * CHUNK, CHUNK), :] = red

    ag_descs = []
    for k in range(1, N_DEV):
        peer = lax.rem(my + k, N_DEV)
        rdma = pltpu.make_async_remote_copy(
            src_ref=out_ref.at[pl.ds(my * CHUNK, CHUNK), :],
            dst_ref=out_ref.at[pl.ds(my * CHUNK, CHUNK), :],
            send_sem=ag_send.at[k - 1],
            recv_sem=ag_recv.at[k - 1],
            device_id=(peer,),
            device_id_type=pl.DeviceIdType.MESH,
        )
        rdma.start()
        ag_descs.append(rdma)
    for r in ag_descs:
        r.wait_recv()
    for r in ag_descs:
        r.wait_send()

    for c in range(N_DEV):

        @pl.when(my != c)
        def _(c=c):
            _rs_desc(pbuf, rs_buf, rs_send, rs_recv, my, c).wait_send()


def kernel(x, Wq, K_ext, V_ext, Wo):
    out = pl.pallas_call(
        _fused_body,
        out_shape=jax.ShapeDtypeStruct((SQ, D), jnp.float32),
        in_specs=[
            pl.BlockSpec(memory_space=pltpu.VMEM),
            pl.BlockSpec(memory_space=pltpu.VMEM),
            pl.BlockSpec(memory_space=pl.ANY),
            pl.BlockSpec(memory_space=pl.ANY),
            pl.BlockSpec(memory_space=pltpu.VMEM),
        ],
        out_specs=pl.BlockSpec(memory_space=pltpu.VMEM),
        scratch_shapes=[
            pltpu.VMEM((HQ_LOCAL, SQ, DH), jnp.float32),
            pltpu.VMEM((HQ_LOCAL, SQ, DH), jnp.float32),
            pltpu.VMEM((SQ, D), jnp.float32),
            pltpu.VMEM((N_DEV, CHUNK, D), jnp.float32),
            pltpu.SemaphoreType.DMA((2 * HQ_LOCAL,)),
            pltpu.SemaphoreType.DMA((N_DEV,)),
            pltpu.SemaphoreType.DMA((N_DEV,)),
            pltpu.SemaphoreType.DMA((N_DEV - 1,)),
            pltpu.SemaphoreType.DMA((N_DEV - 1,)),
        ],
    )(x[0], Wq, K_ext[0], V_ext[0], Wo)

    return out.reshape(1, SQ, D)


# device time: 92676 ns/iter; 38.1079x vs baseline; 1.5372x over previous
import jax
import jax.numpy as jnp
from jax import lax
from jax.experimental import pallas as pl
from jax.experimental.pallas import tpu as pltpu

N_DEV = 32
SQ = 1024
D = 1024
HQ = 256
HQ_LOCAL = 8
DH = 128
BLK = 64
SCALE = 0.08838834764831843
CHUNK = SQ // N_DEV
NBLK = 4
RB = SQ // NBLK
CPB = RB // CHUNK


def _rs_desc(pbuf, rs_buf, rs_send, rs_recv, my, c):
    return pltpu.make_async_remote_copy(
        src_ref=pbuf.at[pl.ds(c * CHUNK, CHUNK), :],
        dst_ref=rs_buf.at[my],
        send_sem=rs_send.at[c],
        recv_sem=rs_recv.at[my],
        device_id=(c,),
        device_id_type=pl.DeviceIdType.MESH,
    )


def _fused_body(x_ref, wq_ref, k_any, v_any, wo_ref, out_ref,
                k_vmem, v_vmem, pbuf, rs_buf, ag_buf,
                kv_sems, rs_send, rs_recv, ag_send, ag_recv):
    my = lax.axis_index("i")
    h0 = my * HQ_LOCAL

    kv_copies = []
    for h in range(HQ_LOCAL):
        ck = pltpu.make_async_copy(
            k_any.at[:, h0 + h, :], k_vmem.at[h], kv_sems.at[h])
        cv = pltpu.make_async_copy(
            v_any.at[:, h0 + h, :], v_vmem.at[h], kv_sems.at[HQ_LOCAL + h])
        ck.start()
        cv.start()
        kv_copies.append(ck)
        kv_copies.append(cv)

    Q = jnp.dot(x_ref[:, :], wq_ref[:, :], preferred_element_type=jnp.float32)
    Wo = wo_ref[:, :]

    for c in kv_copies:
        c.wait()

    for b in range(NBLK):
        r0 = b * RB
        ncol = (b + 1) * RB
        qblk = (r0 + lax.broadcasted_iota(jnp.int32, (RB, ncol), 0)) // BLK
        kblk = lax.broadcasted_iota(jnp.int32, (RB, ncol), 1) // BLK
        mask = kblk <= qblk
        ctx_parts = []
        for h in range(HQ_LOCAL):
            q_h = Q[r0:r0 + RB, h * DH:(h + 1) * DH]
            s = lax.dot_general(
                q_h, k_vmem[h][:ncol, :], (((1,), (1,)), ((), ())),
                preferred_element_type=jnp.float32,
            ) * SCALE
            s = jnp.where(mask, s, -1e9)
            m = jnp.max(s, axis=1, keepdims=True)
            w = jnp.exp(s - m)
            w = w / jnp.sum(w, axis=1, keepdims=True)
            ctx_parts.append(
                jnp.dot(w, v_vmem[h][:ncol, :],
                        preferred_element_type=jnp.float32))
        ctx_b = jnp.concatenate(ctx_parts, axis=1)
        pb = jnp.dot(ctx_b, Wo, preferred_element_type=jnp.float32)
        pbuf[pl.ds(r0, RB), :] = pb.astype(jnp.bfloat16)

        for j in range(CPB):
            c = b * CPB + j

            @pl.when(my != c)
            def _(c=c):
                _rs_desc(pbuf, rs_buf, rs_send, rs_recv, my, c).start()

    pbuf_my = pbuf[pl.ds(my * CHUNK, CHUNK), :]
    rs_buf[pl.ds(my, 1)] = pbuf_my[None, :, :]

    for k in range(1, N_DEV):
        s = lax.rem(my + k, N_DEV)
        pltpu.make_async_remote_copy(
            src_ref=pbuf.at[pl.ds(0, CHUNK), :],
            dst_ref=rs_buf.at[s],
            send_sem=rs_send.at[0],
            recv_sem=rs_recv.at[s],
            device_id=(s,),
            device_id_type=pl.DeviceIdType.MESH,
        ).wait_recv()

    red = jnp.sum(rs_buf[:, :, :].astype(jnp.float32), axis=0)
    ag_buf[pl.ds(my, 1)] = red.astype(jnp.bfloat16)[None, :, :]

    ag_descs = []
    for k in range(1, N_DEV):
        peer = lax.rem(my + k, N_DEV)
        rdma = pltpu.make_async_remote_copy(
            src_ref=ag_buf.at[my],
            dst_ref=ag_buf.at[my],
            send_sem=ag_send.at[k - 1],
            recv_sem=ag_recv.at[my],
            device_id=(peer,),
            device_id_type=pl.DeviceIdType.MESH,
        )
        rdma.start()
        ag_descs.append(rdma)

    for k in range(1, N_DEV):
        s = lax.rem(my + k, N_DEV)
        pltpu.make_async_remote_copy(
            src_ref=ag_buf.at[0],
            dst_ref=ag_buf.at[s],
            send_sem=ag_send.at[0],
            recv_sem=ag_recv.at[s],
            device_id=(s,),
            device_id_type=pl.DeviceIdType.MESH,
        ).wait_recv()

    out_ref[:, :] = ag_buf[:, :, :].reshape(SQ, D).astype(jnp.float32)

    for r in ag_descs:
        r.wait_send()
    for c in range(N_DEV):

        @pl.when(my != c)
        def _(c=c):
            _rs_desc(pbuf, rs_buf, rs_send, rs_recv, my, c).wait_send()


def kernel(x, Wq, K_ext, V_ext, Wo):
    out = pl.pallas_call(
        _fused_body,
        out_shape=jax.ShapeDtypeStruct((SQ, D), jnp.float32),
        in_specs=[
            pl.BlockSpec(memory_space=pltpu.VMEM),
            pl.BlockSpec(memory_space=pltpu.VMEM),
            pl.BlockSpec(memory_space=pl.ANY),
            pl.BlockSpec(memory_space=pl.ANY),
            pl.BlockSpec(memory_space=pltpu.VMEM),
        ],
        out_specs=pl.BlockSpec(memory_space=pltpu.VMEM),
        scratch_shapes=[
            pltpu.VMEM((HQ_LOCAL, SQ, DH), jnp.float32),
            pltpu.VMEM((HQ_LOCAL, SQ, DH), jnp.float32),
            pltpu.VMEM((SQ, D), jnp.bfloat16),
            pltpu.VMEM((N_DEV, CHUNK, D), jnp.bfloat16),
            pltpu.VMEM((N_DEV, CHUNK, D), jnp.bfloat16),
            pltpu.SemaphoreType.DMA((2 * HQ_LOCAL,)),
            pltpu.SemaphoreType.DMA((N_DEV,)),
            pltpu.SemaphoreType.DMA((N_DEV,)),
            pltpu.SemaphoreType.DMA((N_DEV - 1,)),
            pltpu.SemaphoreType.DMA((N_DEV,)),
        ],
    )(x[0], Wq, K_ext[0], V_ext[0], Wo)

    return out.reshape(1, SQ, D)
